# Initial kernel scaffold; baseline (speedup 1.0000x reference)
import jax
import jax.numpy as jnp
from jax.experimental import pallas as pl


def kernel(H, edge_weights):
    raise NotImplementedError("write your pallas kernel here")


# speedup vs baseline: 9.0807x; 9.0807x over previous
import functools

import jax
import jax.numpy as jnp
from jax import lax
from jax.experimental import pallas as pl
from jax.experimental.pallas import tpu as pltpu
from jax.experimental.pallas import tpu_sc as plsc

N = 10000
NP_ = 10240
E = 320000
HD = 128
NC = 2
NS = 16
L = 16
NW = NC * NS
EPW = E // NW
BLK = 80
NBLK = EPW // BLK
ROWS_PER_TILE = NP_ // NS


def _lane_bcast(vec, lane):
    idx = jnp.full((L, 1), lane, jnp.int32)
    dnums = lax.GatherDimensionNumbers(
        offset_dims=(), collapsed_slice_dims=(0,), start_index_map=(0,))
    return lax.gather(vec, idx, dnums, slice_sizes=(1,),
                      mode=lax.GatherScatterMode.PROMISE_IN_BOUNDS)


def _sc_partials(H2d, src, dst, w):
    mesh = plsc.VectorSubcoreMesh(core_axis_name="c", subcore_axis_name="s")

    @functools.partial(
        pl.kernel,
        mesh=mesh,
        out_type=jax.ShapeDtypeStruct((NC, NP_, HD), jnp.float32),
        scratch_types=[
            pltpu.VMEM((BLK,), jnp.int32),
            pltpu.VMEM((BLK,), jnp.int32),
            pltpu.VMEM((BLK,), jnp.float32),
            pltpu.VMEM((BLK, HD), jnp.float32),
            pltpu.VMEM((128, HD), jnp.float32),
            pltpu.VMEM_SHARED((NP_, HD), jnp.float32),
            pltpu.SemaphoreType.DMA,
        ],
    )
    def k(src_hbm, dst_hbm, w_hbm, h_hbm, out_hbm,
          src_v, dst_v, w_v, rows_v, z_v, acc, sem):
        cid = lax.axis_index("c")
        sid = lax.axis_index("s")
        wid = cid * NS + sid

        zero16 = jnp.zeros((L,), jnp.float32)

        def zfill(r, _):
            for j in range(HD // L):
                z_v[r, pl.ds(j * L, L)] = zero16
            return 0

        lax.fori_loop(0, 128, zfill, 0)
        for i in range(5):
            pltpu.sync_copy(z_v, acc.at[pl.ds(sid * ROWS_PER_TILE + i * 128, 128)])
        plsc.subcore_barrier()

        base_edge = wid * EPW

        def block_body(b, _):
            base = base_edge + b * BLK
            pltpu.sync_copy(src_hbm.at[pl.ds(base, BLK)], src_v)
            pltpu.sync_copy(dst_hbm.at[pl.ds(base, BLK)], dst_v)
            pltpu.sync_copy(w_hbm.at[pl.ds(base, BLK)], w_v)
            pltpu.async_copy(h_hbm.at[dst_v], rows_v, sem).wait()

            def scale(g, _):
                wv = w_v[pl.ds(g * L, L)]
                for e in range(L):
                    wb = _lane_bcast(wv, e)
                    row = g * L + e
                    for j in range(HD // L):
                        sl = pl.ds(j * L, L)
                        rows_v[row, sl] = rows_v[row, sl] * wb
                return 0

            lax.fori_loop(0, BLK // L, scale, 0)
            pltpu.sync_copy(rows_v, acc.at[src_v], add=True)
            return 0

        lax.fori_loop(0, NBLK, block_body, 0)
        plsc.subcore_barrier()

        row0 = sid * ROWS_PER_TILE
        pltpu.sync_copy(acc.at[pl.ds(row0, ROWS_PER_TILE)],
                        out_hbm.at[cid, pl.ds(row0, ROWS_PER_TILE)])

    return k(src, dst, w, H2d)


def _tc_add(partials):
    def body(p_ref, o_ref):
        o_ref[...] = p_ref[0] + p_ref[1]

    return pl.pallas_call(
        body,
        grid=(10,),
        in_specs=[pl.BlockSpec((NC, NP_ // 10, HD), lambda i: (0, i, 0))],
        out_specs=pl.BlockSpec((NP_ // 10, HD), lambda i: (i, 0)),
        out_shape=jax.ShapeDtypeStruct((NP_, HD), jnp.float32),
    )(partials)


@jax.jit
def kernel(H, edge_weights):
    H2d = H[0]
    src = edge_weights[0, :, 0].astype(jnp.int32)
    dst = edge_weights[0, :, 1].astype(jnp.int32)
    w = edge_weights[0, :, 2]
    partials = _sc_partials(H2d, src, dst, w)
    return _tc_add(partials)[:N][None]
